# baseline (speedup 1.0000x reference)
import jax
import jax.numpy as jnp
from jax.experimental import pallas as pl
from jax.experimental.pallas import tpu as pltpu

_B = 64; _L = 80; _M = 3; _G = 128; _WP = 8; _WF = 8
_H = 2; _DH = 128; _HC = 128; _TAG = 6
_A_DIM = 100; _T_DIM = 768; _V_DIM = 512
_D = 8
_DL = _D * _L
_CL = 2 * _L
_NC = _D // 2
_CROSS_IN = {0: ((1, 11), (2, 13)), 1: ((0, 9), (2, 14)), 2: ((0, 10), (1, 12))}


def _net_kernel(audio_ref, text_ref, vis_ref, spk_ref,
                Wa_ref, ba_ref, Wt_ref, bt_ref, Wv_ref, bv_ref,
                se_ref, WX_ref, Wpast_ref, Wfut_ref, Wself_ref, br_ref,
                Wqkv_ref, W1_ref, b1_ref, W2_ref, b2_ref,
                out_ref):
    f32 = jnp.float32
    bf16 = jnp.bfloat16

    def dot(a, b):
        return jax.lax.dot_general(a, b, (((1,), (0,)), ((), ())),
                                   preferred_element_type=f32)

    def dot_t(a, b):
        return jax.lax.dot_general(a, b, (((1,), (1,)), ((), ())),
                                   preferred_element_type=f32)

    WX = WX_ref[:, :].astype(bf16)
    Wpast = Wpast_ref[:, :, :].astype(bf16)
    Wfut = Wfut_ref[:, :, :].astype(bf16)
    Wself = Wself_ref[:, :].astype(bf16)
    Wqkv = Wqkv_ref[:, :].astype(bf16)
    W1 = W1_ref[:, :].astype(bf16)
    W2 = W2_ref[:, :].astype(bf16)

    a = jnp.maximum(dot(audio_ref[0].astype(bf16), Wa_ref[:, :].astype(bf16))
                    + ba_ref[:, :], 0.0)
    t = jnp.maximum(dot(text_ref[0].astype(bf16), Wt_ref[:, :].astype(bf16))
                    + bt_ref[:, :], 0.0)
    v = jnp.maximum(dot(vis_ref[0].astype(bf16), Wv_ref[:, :].astype(bf16))
                    + bv_ref[:, :], 0.0)
    s = spk_ref[0]
    emb = se_ref[0:1, :] + s * (se_ref[1:2, :] - se_ref[0:1, :])
    X_full = jnp.concatenate([a + emb, t + emb, v + emb], axis=1)
    Xb_full = X_full.astype(bf16)

    ti = jax.lax.broadcasted_iota(jnp.int32, (_CL, _CL), 0)
    tj = jax.lax.broadcasted_iota(jnp.int32, (_CL, _CL), 1)
    same = (ti // _L) == (tj // _L)
    off = tj - ti
    past = jnp.where(same & (off >= -_WP) & (off <= -1), 1.0, 0.0)
    fut = jnp.where(same & (off >= 1) & (off <= _WF), 1.0, 0.0)
    pf = jnp.concatenate([past, fut], axis=0).astype(bf16)
    in_band = same & (off >= -_WP) & (off <= _WF)
    bias = jnp.where(in_band, 0.0, -1e30).astype(f32)
    deg = (jnp.sum(past, axis=1, keepdims=True)
           + jnp.sum(fut, axis=1, keepdims=True) + 3.0)
    inv_deg = 1.0 / deg

    for c in range(_NC):
        rows = slice(c * _CL, (c + 1) * _CL)
        Xb = Xb_full[rows]

        bandsum = dot(pf, Xb).astype(bf16)
        aggx = dot(Xb, WX)
        hs = []
        for m in range(_M):
            col = slice(m * _G, (m + 1) * _G)
            am = (aggx[:, col]
                  + dot(bandsum[0:_CL, col], Wpast[m])
                  + dot(bandsum[_CL:2 * _CL, col], Wfut[m]))
            sm = dot(Xb[:, col], Wself)
            hs.append(jnp.maximum(am * inv_deg + sm + br_ref[:, :], 0.0))
        h1 = jnp.concatenate(hs, axis=0)

        qkv = dot(h1.astype(bf16), Wqkv)
        qkv_b = qkv.astype(bf16)
        outs = []
        for m in range(_M):
            (j1, _), (j2, _) = _CROSS_IN[m]
            row = slice(m * _CL, (m + 1) * _CL)
            r1 = slice(j1 * _CL, (j1 + 1) * _CL)
            r2 = slice(j2 * _CL, (j2 + 1) * _CL)
            head_outs = []
            for hh in range(_H):
                qc = slice(hh * _DH, (hh + 1) * _DH)
                kc = slice(_H * _DH + hh * _DH, _H * _DH + (hh + 1) * _DH)
                vc = slice(2 * _H * _DH + hh * _DH,
                           2 * _H * _DH + (hh + 1) * _DH)
                sc = dot_t(qkv_b[row, qc], qkv_b[row, kc]) + bias
                c1 = jnp.sum(qkv[row, qc] * qkv[r1, kc], axis=1, keepdims=True)
                c2 = jnp.sum(qkv[row, qc] * qkv[r2, kc], axis=1, keepdims=True)
                rmax = jnp.maximum(jnp.max(sc, axis=1, keepdims=True),
                                   jnp.maximum(c1, c2))
                eb = jnp.exp(sc - rmax)
                e1 = jnp.exp(c1 - rmax); e2 = jnp.exp(c2 - rmax)
                inv_den = 1.0 / (jnp.sum(eb, axis=1, keepdims=True) + e1 + e2)
                o = (dot(eb.astype(bf16), qkv_b[row, vc]) * inv_den
                     + (e1 * inv_den) * qkv[r1, vc]
                     + (e2 * inv_den) * qkv[r2, vc])
                head_outs.append(o)
            outs.append(jnp.concatenate(head_outs, axis=1))

        feat = jnp.concatenate(outs, axis=1).astype(bf16)
        hid = jnp.maximum(dot(feat, W1) + b1_ref[:, :], 0.0)
        out_ref[0, rows] = dot(hid.astype(bf16), W2) + b2_ref[:, :]


def kernel(audio_tensor, text_tensor, visual_tensor, Wa, ba, Wt, bt, Wv, bv,
           spk_emb, Wr, Wself, br, Wq, Wk, Wv2, W1, b1, W2, b2,
           speaker_tensor, text_len_tensor, edge_index, edge_type):
    f32 = jnp.float32
    Bn, Ln = speaker_tensor.shape
    nblk = Bn // _D
    spk_f = speaker_tensor.astype(f32).reshape(nblk, _DL, 1)

    cols = []
    for m in range(_M):
        blocks = [None, None, None]
        blocks[m] = Wr[3 * m + 1]
        for j, r in _CROSS_IN[m]:
            blocks[j] = Wr[r]
        cols.append(jnp.concatenate(blocks, axis=0))
    WX = jnp.concatenate(cols, axis=1)
    Wpast = jnp.stack([Wr[0], Wr[3], Wr[6]])
    Wfut = jnp.stack([Wr[2], Wr[5], Wr[8]])
    scale = 1.0 / (_DH ** 0.5)
    Wqkv = jnp.concatenate([Wq * scale, Wk, Wv2], axis=1)

    def row2(x):
        return x.reshape(1, -1).astype(f32)

    def fixed(shape):
        nd = len(shape)
        return pl.BlockSpec(shape, lambda i, _n=nd: (0,) * _n)

    out = pl.pallas_call(
        _net_kernel,
        grid=(nblk,),
        in_specs=[
            pl.BlockSpec((1, _DL, _A_DIM), lambda i: (i, 0, 0)),
            pl.BlockSpec((1, _DL, _T_DIM), lambda i: (i, 0, 0)),
            pl.BlockSpec((1, _DL, _V_DIM), lambda i: (i, 0, 0)),
            pl.BlockSpec((1, _DL, 1), lambda i: (i, 0, 0)),
            fixed((_A_DIM, _G)), fixed((1, _G)),
            fixed((_T_DIM, _G)), fixed((1, _G)),
            fixed((_V_DIM, _G)), fixed((1, _G)),
            fixed((2, _G)),
            fixed((3 * _G, 3 * _G)),
            fixed((_M, _G, _G)), fixed((_M, _G, _G)),
            fixed((_G, _G)), fixed((1, _G)),
            fixed((_G, 3 * _H * _DH)),
            fixed((_M * _H * _DH, _HC)), fixed((1, _HC)),
            fixed((_HC, _TAG)), fixed((1, _TAG)),
        ],
        out_specs=pl.BlockSpec((1, _DL, _TAG), lambda i: (i, 0, 0)),
        out_shape=jax.ShapeDtypeStruct((nblk, _DL, _TAG), f32),
        compiler_params=pltpu.CompilerParams(
            dimension_semantics=("parallel",)),
    )(audio_tensor.reshape(nblk, _DL, _A_DIM),
      text_tensor.reshape(nblk, _DL, _T_DIM),
      visual_tensor.reshape(nblk, _DL, _V_DIM), spk_f,
      Wa, row2(ba), Wt, row2(bt), Wv, row2(bv),
      spk_emb, WX, Wpast, Wfut, Wself, row2(br),
      Wqkv, W1, row2(b1), W2, b2.reshape(1, -1).astype(f32))
    return out.reshape(Bn * Ln, _TAG)


# speedup vs baseline: 1.0244x; 1.0244x over previous
import jax
import jax.numpy as jnp
from jax.experimental import pallas as pl
from jax.experimental.pallas import tpu as pltpu

_B = 64; _L = 80; _M = 3; _G = 128; _WP = 8; _WF = 8
_H = 2; _DH = 128; _HC = 128; _TAG = 6
_A_DIM = 100; _T_DIM = 768; _V_DIM = 512
_D = 4
_DL = _D * _L
_CL = 2 * _L
_NC = _D // 2
_CROSS_IN = {0: ((1, 11), (2, 13)), 1: ((0, 9), (2, 14)), 2: ((0, 10), (1, 12))}


def _net_kernel(audio_ref, text_ref, vis_ref, spk_ref,
                Wa_ref, ba_ref, Wt_ref, bt_ref, Wv_ref, bv_ref,
                se_ref, Wr_ref, Wself_ref, br_ref,
                Wqkv_ref, W1_ref, b1_ref, W2p_ref, b2p_ref,
                out_ref):
    f32 = jnp.float32
    bf16 = jnp.bfloat16

    def dot(a, b):
        return jax.lax.dot_general(a.astype(bf16), b.astype(bf16),
                                   (((1,), (0,)), ((), ())),
                                   preferred_element_type=f32)

    def dot_t(a, b):
        return jax.lax.dot_general(a.astype(bf16), b.astype(bf16),
                                   (((1,), (1,)), ((), ())),
                                   preferred_element_type=f32)

    a = jnp.maximum(dot(audio_ref[0], Wa_ref[:, :]) + ba_ref[:, :], 0.0)
    t = jnp.maximum(dot(text_ref[0], Wt_ref[:, :]) + bt_ref[:, :], 0.0)
    v = jnp.maximum(dot(vis_ref[0], Wv_ref[:, :]) + bv_ref[:, :], 0.0)
    s = spk_ref[0]
    emb = se_ref[0:1, :] + s * (se_ref[1:2, :] - se_ref[0:1, :])
    a = a + emb; t = t + emb; v = v + emb
    x_all = jnp.concatenate([a, t, v], axis=0)
    self_term = dot(x_all, Wself_ref[:, :])

    ti = jax.lax.broadcasted_iota(jnp.int32, (_CL, _CL), 0)
    tj = jax.lax.broadcasted_iota(jnp.int32, (_CL, _CL), 1)
    same = (ti // _L) == (tj // _L)
    off = tj - ti
    past = jnp.where(same & (off >= -_WP) & (off <= -1), 1.0, 0.0).astype(f32)
    fut = jnp.where(same & (off >= 1) & (off <= _WF), 1.0, 0.0).astype(f32)
    band = same & (off >= -_WP) & (off <= _WF)
    deg = (jnp.sum(past, axis=1, keepdims=True)
           + jnp.sum(fut, axis=1, keepdims=True) + 3.0)
    inv_deg = 1.0 / deg
    scale = 1.0 / (_DH ** 0.5)

    for c in range(_NC):
        rows = slice(c * _CL, (c + 1) * _CL)
        xs = (a[rows], t[rows], v[rows])

        hs = []
        for m in range(_M):
            pm = dot(past, xs[m])
            fm = dot(fut, xs[m])
            agg = (dot(pm, Wr_ref[3 * m]) + dot(xs[m], Wr_ref[3 * m + 1])
                   + dot(fm, Wr_ref[3 * m + 2]))
            for j, r in _CROSS_IN[m]:
                agg = agg + dot(xs[j], Wr_ref[r])
            h = (agg * inv_deg + br_ref[:, :]
                 + self_term[m * _DL + c * _CL:m * _DL + (c + 1) * _CL])
            hs.append(jnp.maximum(h, 0.0))
        h1 = jnp.concatenate(hs, axis=0)

        qkv = dot(h1, Wqkv_ref[:, :])
        outs = []
        for m in range(_M):
            (j1, _), (j2, _) = _CROSS_IN[m]
            row = slice(m * _CL, (m + 1) * _CL)
            r1 = slice(j1 * _CL, (j1 + 1) * _CL)
            r2 = slice(j2 * _CL, (j2 + 1) * _CL)
            head_outs = []
            for hh in range(_H):
                qc = slice(hh * _DH, (hh + 1) * _DH)
                kc = slice(_H * _DH + hh * _DH, _H * _DH + (hh + 1) * _DH)
                vc = slice(2 * _H * _DH + hh * _DH,
                           2 * _H * _DH + (hh + 1) * _DH)
                qm = qkv[row, qc] * scale
                sc = dot_t(qm, qkv[row, kc])
                c1 = jnp.sum(qm * qkv[r1, kc], axis=1, keepdims=True)
                c2 = jnp.sum(qm * qkv[r2, kc], axis=1, keepdims=True)
                masked = jnp.where(band, sc, -1e30)
                rmax = jnp.maximum(jnp.max(masked, axis=1, keepdims=True),
                                   jnp.maximum(c1, c2))
                eb = jnp.where(band, jnp.exp(sc - rmax), 0.0)
                e1 = jnp.exp(c1 - rmax); e2 = jnp.exp(c2 - rmax)
                inv_den = 1.0 / (jnp.sum(eb, axis=1, keepdims=True) + e1 + e2)
                o = (dot(eb, qkv[row, vc]) * inv_den
                     + (e1 * inv_den) * qkv[r1, vc]
                     + (e2 * inv_den) * qkv[r2, vc])
                head_outs.append(o)
            outs.append(jnp.concatenate(head_outs, axis=1))

        feat = jnp.concatenate(outs, axis=1)
        hid = jnp.maximum(dot(feat, W1_ref[:, :]) + b1_ref[:, :], 0.0)
        logits = dot(hid, W2p_ref[:, :]) + b2p_ref[:, :]
        out_ref[0, rows] = logits[:, :_TAG]


def kernel(audio_tensor, text_tensor, visual_tensor, Wa, ba, Wt, bt, Wv, bv,
           spk_emb, Wr, Wself, br, Wq, Wk, Wv2, W1, b1, W2, b2,
           speaker_tensor, text_len_tensor, edge_index, edge_type):
    f32 = jnp.float32
    Bn, Ln = speaker_tensor.shape
    nblk = Bn // _D
    spk_f = speaker_tensor.astype(f32).reshape(nblk, _DL, 1)
    W2p = jnp.zeros((_HC, 128), f32).at[:, :_TAG].set(W2)
    b2p = jnp.zeros((1, 128), f32).at[0, :_TAG].set(b2)
    Wqkv = jnp.concatenate([Wq, Wk, Wv2], axis=1)

    def row2(x):
        return x.reshape(1, -1).astype(f32)

    def fixed(shape):
        nd = len(shape)
        return pl.BlockSpec(shape, lambda i, _n=nd: (0,) * _n)

    out = pl.pallas_call(
        _net_kernel,
        grid=(nblk,),
        in_specs=[
            pl.BlockSpec((1, _DL, _A_DIM), lambda i: (i, 0, 0)),
            pl.BlockSpec((1, _DL, _T_DIM), lambda i: (i, 0, 0)),
            pl.BlockSpec((1, _DL, _V_DIM), lambda i: (i, 0, 0)),
            pl.BlockSpec((1, _DL, 1), lambda i: (i, 0, 0)),
            fixed((_A_DIM, _G)), fixed((1, _G)),
            fixed((_T_DIM, _G)), fixed((1, _G)),
            fixed((_V_DIM, _G)), fixed((1, _G)),
            fixed((2, _G)),
            fixed((15, _G, _G)),
            fixed((_G, _G)), fixed((1, _G)),
            fixed((_G, 3 * _H * _DH)),
            fixed((_M * _H * _DH, _HC)), fixed((1, _HC)),
            fixed((_HC, 128)), fixed((1, 128)),
        ],
        out_specs=pl.BlockSpec((1, _DL, _TAG), lambda i: (i, 0, 0)),
        out_shape=jax.ShapeDtypeStruct((nblk, _DL, _TAG), f32),
        compiler_params=pltpu.CompilerParams(
            dimension_semantics=("parallel",)),
    )(audio_tensor.reshape(nblk, _DL, _A_DIM),
      text_tensor.reshape(nblk, _DL, _T_DIM),
      visual_tensor.reshape(nblk, _DL, _V_DIM), spk_f,
      Wa, row2(ba), Wt, row2(bt), Wv, row2(bv),
      spk_emb, Wr, Wself, row2(br), Wqkv, W1, row2(b1), W2p, b2p)
    return out.reshape(Bn * Ln, _TAG)


# speedup vs baseline: 1.0518x; 1.0267x over previous
import jax
import jax.numpy as jnp
from jax.experimental import pallas as pl
from jax.experimental.pallas import tpu as pltpu

_B = 64; _L = 80; _M = 3; _G = 128; _WP = 8; _WF = 8
_H = 2; _DH = 128; _HC = 128; _TAG = 6
_A_DIM = 100; _T_DIM = 768; _V_DIM = 512
_D = 8
_DL = _D * _L
_CL = 2 * _L
_NC = _D // 2
_CROSS_IN = {0: ((1, 11), (2, 13)), 1: ((0, 9), (2, 14)), 2: ((0, 10), (1, 12))}


def _net_kernel(audio_ref, text_ref, vis_ref, spk_ref,
                Wa_ref, ba_ref, Wt_ref, bt_ref, Wv_ref, bv_ref,
                se_ref, Wr_ref, Wself_ref, br_ref,
                Wqkv_ref, W1_ref, b1_ref, W2p_ref, b2p_ref,
                out_ref):
    f32 = jnp.float32
    bf16 = jnp.bfloat16

    def dot(a, b):
        return jax.lax.dot_general(a.astype(bf16), b.astype(bf16),
                                   (((1,), (0,)), ((), ())),
                                   preferred_element_type=f32)

    def dot_t(a, b):
        return jax.lax.dot_general(a.astype(bf16), b.astype(bf16),
                                   (((1,), (1,)), ((), ())),
                                   preferred_element_type=f32)

    a = jnp.maximum(dot(audio_ref[0], Wa_ref[:, :]) + ba_ref[:, :], 0.0)
    t = jnp.maximum(dot(text_ref[0], Wt_ref[:, :]) + bt_ref[:, :], 0.0)
    v = jnp.maximum(dot(vis_ref[0], Wv_ref[:, :]) + bv_ref[:, :], 0.0)
    s = spk_ref[0]
    emb = se_ref[0:1, :] + s * (se_ref[1:2, :] - se_ref[0:1, :])
    a = a + emb; t = t + emb; v = v + emb
    x_all = jnp.concatenate([a, t, v], axis=0)
    self_term = dot(x_all, Wself_ref[:, :])

    ti = jax.lax.broadcasted_iota(jnp.int32, (_CL, _CL), 0)
    tj = jax.lax.broadcasted_iota(jnp.int32, (_CL, _CL), 1)
    same = (ti // _L) == (tj // _L)
    off = tj - ti
    past = jnp.where(same & (off >= -_WP) & (off <= -1), 1.0, 0.0).astype(f32)
    fut = jnp.where(same & (off >= 1) & (off <= _WF), 1.0, 0.0).astype(f32)
    band = same & (off >= -_WP) & (off <= _WF)
    deg = (jnp.sum(past, axis=1, keepdims=True)
           + jnp.sum(fut, axis=1, keepdims=True) + 3.0)
    inv_deg = 1.0 / deg
    scale = 1.0 / (_DH ** 0.5)

    for c in range(_NC):
        rows = slice(c * _CL, (c + 1) * _CL)
        xs = (a[rows], t[rows], v[rows])

        hs = []
        for m in range(_M):
            pm = dot(past, xs[m])
            fm = dot(fut, xs[m])
            agg = (dot(pm, Wr_ref[3 * m]) + dot(xs[m], Wr_ref[3 * m + 1])
                   + dot(fm, Wr_ref[3 * m + 2]))
            for j, r in _CROSS_IN[m]:
                agg = agg + dot(xs[j], Wr_ref[r])
            h = (agg * inv_deg + br_ref[:, :]
                 + self_term[m * _DL + c * _CL:m * _DL + (c + 1) * _CL])
            hs.append(jnp.maximum(h, 0.0))
        h1 = jnp.concatenate(hs, axis=0)

        qkv = dot(h1, Wqkv_ref[:, :])
        outs = []
        for m in range(_M):
            (j1, _), (j2, _) = _CROSS_IN[m]
            row = slice(m * _CL, (m + 1) * _CL)
            r1 = slice(j1 * _CL, (j1 + 1) * _CL)
            r2 = slice(j2 * _CL, (j2 + 1) * _CL)
            head_outs = []
            for hh in range(_H):
                qc = slice(hh * _DH, (hh + 1) * _DH)
                kc = slice(_H * _DH + hh * _DH, _H * _DH + (hh + 1) * _DH)
                vc = slice(2 * _H * _DH + hh * _DH,
                           2 * _H * _DH + (hh + 1) * _DH)
                qm = qkv[row, qc] * scale
                sc = dot_t(qm, qkv[row, kc])
                c1 = jnp.sum(qm * qkv[r1, kc], axis=1, keepdims=True)
                c2 = jnp.sum(qm * qkv[r2, kc], axis=1, keepdims=True)
                masked = jnp.where(band, sc, -1e30)
                rmax = jnp.maximum(jnp.max(masked, axis=1, keepdims=True),
                                   jnp.maximum(c1, c2))
                eb = jnp.where(band, jnp.exp(sc - rmax), 0.0)
                e1 = jnp.exp(c1 - rmax); e2 = jnp.exp(c2 - rmax)
                inv_den = 1.0 / (jnp.sum(eb, axis=1, keepdims=True) + e1 + e2)
                o = (dot(eb, qkv[row, vc]) * inv_den
                     + (e1 * inv_den) * qkv[r1, vc]
                     + (e2 * inv_den) * qkv[r2, vc])
                head_outs.append(o)
            outs.append(jnp.concatenate(head_outs, axis=1))

        feat = jnp.concatenate(outs, axis=1)
        hid = jnp.maximum(dot(feat, W1_ref[:, :]) + b1_ref[:, :], 0.0)
        logits = dot(hid, W2p_ref[:, :]) + b2p_ref[:, :]
        out_ref[0, rows] = logits[:, :_TAG]


def kernel(audio_tensor, text_tensor, visual_tensor, Wa, ba, Wt, bt, Wv, bv,
           spk_emb, Wr, Wself, br, Wq, Wk, Wv2, W1, b1, W2, b2,
           speaker_tensor, text_len_tensor, edge_index, edge_type):
    f32 = jnp.float32
    Bn, Ln = speaker_tensor.shape
    nblk = Bn // _D
    spk_f = speaker_tensor.astype(f32).reshape(nblk, _DL, 1)
    W2p = jnp.zeros((_HC, 128), f32).at[:, :_TAG].set(W2)
    b2p = jnp.zeros((1, 128), f32).at[0, :_TAG].set(b2)
    Wqkv = jnp.concatenate([Wq, Wk, Wv2], axis=1)

    def row2(x):
        return x.reshape(1, -1).astype(f32)

    def fixed(shape):
        nd = len(shape)
        return pl.BlockSpec(shape, lambda i, _n=nd: (0,) * _n)

    out = pl.pallas_call(
        _net_kernel,
        grid=(nblk,),
        in_specs=[
            pl.BlockSpec((1, _DL, _A_DIM), lambda i: (i, 0, 0)),
            pl.BlockSpec((1, _DL, _T_DIM), lambda i: (i, 0, 0)),
            pl.BlockSpec((1, _DL, _V_DIM), lambda i: (i, 0, 0)),
            pl.BlockSpec((1, _DL, 1), lambda i: (i, 0, 0)),
            fixed((_A_DIM, _G)), fixed((1, _G)),
            fixed((_T_DIM, _G)), fixed((1, _G)),
            fixed((_V_DIM, _G)), fixed((1, _G)),
            fixed((2, _G)),
            fixed((15, _G, _G)),
            fixed((_G, _G)), fixed((1, _G)),
            fixed((_G, 3 * _H * _DH)),
            fixed((_M * _H * _DH, _HC)), fixed((1, _HC)),
            fixed((_HC, 128)), fixed((1, 128)),
        ],
        out_specs=pl.BlockSpec((1, _DL, _TAG), lambda i: (i, 0, 0)),
        out_shape=jax.ShapeDtypeStruct((nblk, _DL, _TAG), f32),
        compiler_params=pltpu.CompilerParams(
            dimension_semantics=("parallel",)),
    )(audio_tensor.reshape(nblk, _DL, _A_DIM),
      text_tensor.reshape(nblk, _DL, _T_DIM),
      visual_tensor.reshape(nblk, _DL, _V_DIM), spk_f,
      Wa, row2(ba), Wt, row2(bt), Wv, row2(bv),
      spk_emb, Wr, Wself, row2(br), Wqkv, W1, row2(b1), W2p, b2p)
    return out.reshape(Bn * Ln, _TAG)
